# Initial kernel scaffold; baseline (speedup 1.0000x reference)
import jax
import jax.numpy as jnp
from jax.experimental import pallas as pl


def kernel(x, pos, params, edge_index):
    raise NotImplementedError("write your pallas kernel here")


# speedup vs baseline: 1.2058x; 1.2058x over previous
import functools

import jax
import jax.numpy as jnp
from jax.experimental import pallas as pl
from jax.experimental.pallas import tpu as pltpu

N = 10000
E = 320000
H = 128
NP = 10240
BE = 2000
BN = 1024

_INTERPRET = False


def _ln(u, g, beta):
    mu = jnp.mean(u, axis=-1, keepdims=True)
    var = jnp.mean((u - mu) * (u - mu), axis=-1, keepdims=True)
    return (u - mu) * jax.lax.rsqrt(var + 1e-5) * g + beta


def _dot(a, b):
    return jnp.dot(a, b, preferred_element_type=jnp.float32)


def _edge_mlp_body(gA, gB, e, w1c, b1, w2, b2, g, beta, out):
    pre = gA[...] + gB[...] + _dot(e[...], w1c[...]) + b1[...]
    t = jnp.maximum(pre, 0.0)
    u = _dot(t, w2[...]) + b2[...]
    out[...] = _ln(u, g[...], beta[...]) + e[...]


def _node_body(h, agg, wa, wb, b1, w2, b2, g, beta, wna, wnb,
               hout, haout, hbout):
    pre = _dot(h[...], wa[...]) + _dot(agg[...], wb[...]) + b1[...]
    t = jnp.maximum(pre, 0.0)
    u = _dot(t, w2[...]) + b2[...]
    hn = h[...] + _ln(u, g[...], beta[...])
    hout[...] = hn
    haout[...] = _dot(hn, wna[...])
    hbout[...] = _dot(hn, wnb[...])


def _node_last_body(h, agg, wa, wb, b1, w2, b2, g, beta,
                    wd1, bd1, wd2, bd2, out):
    pre = _dot(h[...], wa[...]) + _dot(agg[...], wb[...]) + b1[...]
    t = jnp.maximum(pre, 0.0)
    u = _dot(t, w2[...]) + b2[...]
    hn = h[...] + _ln(u, g[...], beta[...])
    d = jnp.maximum(_dot(hn, wd1[...]) + bd1[...], 0.0)
    out[...] = _dot(d, wd2[...]) + bd2[...]


def _enc_nodes_body(x, w1, b1, w2, b2, g, beta, wna, wnb,
                    hout, haout, hbout):
    pre = _dot(x[...], w1[...]) + b1[...]
    t = jnp.maximum(pre, 0.0)
    u = _dot(t, w2[...]) + b2[...]
    hn = _ln(u, g[...], beta[...])
    hout[...] = hn
    haout[...] = _dot(hn, wna[...])
    hbout[...] = _dot(hn, wnb[...])


def _edge_enc_body(pc, pr, w1, b1, w2, b2, g, beta, out):
    d = pc[...] - pr[...]
    r0 = d[:, 0:1]
    r1 = d[:, 1:2]
    dist = jnp.sqrt(r0 * r0 + r1 * r1)
    w = w1[...]
    pre = r0 * w[0:1, :] + r1 * w[1:2, :] + dist * w[2:3, :] + b1[...]
    t = jnp.maximum(pre, 0.0)
    u = _dot(t, w2[...]) + b2[...]
    out[...] = _ln(u, g[...], beta[...])


def _eblk():
    return pl.BlockSpec((BE, H), lambda i: (i, 0))


def _nblk():
    return pl.BlockSpec((BN, H), lambda i: (i, 0))


def _wblk(shape):
    return pl.BlockSpec(shape, lambda i: tuple(0 for _ in shape))


def _edge_mlp(gA, gB, e, p):
    return pl.pallas_call(
        _edge_mlp_body,
        grid=(E // BE,),
        in_specs=[_eblk(), _eblk(), _eblk(),
                  _wblk((H, H)), _wblk((1, H)), _wblk((H, H)),
                  _wblk((1, H)), _wblk((1, H)), _wblk((1, H))],
        out_specs=_eblk(),
        out_shape=jax.ShapeDtypeStruct((E, H), jnp.float32),
        interpret=_INTERPRET,
    )(gA, gB, e, p['w1'][2 * H:], p['b1'].reshape(1, H), p['w2'],
      p['b2'].reshape(1, H), p['g'].reshape(1, H), p['beta'].reshape(1, H))


def _node_update(h, agg, p, w1_next):
    outs = [jax.ShapeDtypeStruct((NP, H), jnp.float32)] * 3
    return pl.pallas_call(
        _node_body,
        grid=(NP // BN,),
        in_specs=[_nblk(), _nblk(),
                  _wblk((H, H)), _wblk((H, H)), _wblk((1, H)),
                  _wblk((H, H)), _wblk((1, H)), _wblk((1, H)), _wblk((1, H)),
                  _wblk((H, H)), _wblk((H, H))],
        out_specs=[_nblk()] * 3,
        out_shape=outs,
        interpret=_INTERPRET,
    )(h, agg, p['w1'][:H], p['w1'][H:], p['b1'].reshape(1, H), p['w2'],
      p['b2'].reshape(1, H), p['g'].reshape(1, H), p['beta'].reshape(1, H),
      w1_next[:H], w1_next[H:2 * H])


def _node_last(h, agg, p, dec):
    wd2 = jnp.pad(dec['w2'], ((0, 0), (0, H - dec['w2'].shape[1])))
    bd2 = jnp.pad(dec['b2'], (0, H - dec['b2'].shape[0])).reshape(1, H)
    return pl.pallas_call(
        _node_last_body,
        grid=(NP // BN,),
        in_specs=[_nblk(), _nblk(),
                  _wblk((H, H)), _wblk((H, H)), _wblk((1, H)),
                  _wblk((H, H)), _wblk((1, H)), _wblk((1, H)), _wblk((1, H)),
                  _wblk((H, H)), _wblk((1, H)), _wblk((H, H)), _wblk((1, H))],
        out_specs=_nblk(),
        out_shape=jax.ShapeDtypeStruct((NP, H), jnp.float32),
        interpret=_INTERPRET,
    )(h, agg, p['w1'][:H], p['w1'][H:], p['b1'].reshape(1, H), p['w2'],
      p['b2'].reshape(1, H), p['g'].reshape(1, H), p['beta'].reshape(1, H),
      dec['w1'], dec['b1'].reshape(1, H), wd2, bd2)


def _enc_nodes(xP, p, w1_0):
    outs = [jax.ShapeDtypeStruct((NP, H), jnp.float32)] * 3
    return pl.pallas_call(
        _enc_nodes_body,
        grid=(NP // BN,),
        in_specs=[pl.BlockSpec((BN, 4), lambda i: (i, 0)),
                  _wblk((4, H)), _wblk((1, H)), _wblk((H, H)),
                  _wblk((1, H)), _wblk((1, H)), _wblk((1, H)),
                  _wblk((H, H)), _wblk((H, H))],
        out_specs=[_nblk()] * 3,
        out_shape=outs,
        interpret=_INTERPRET,
    )(xP, p['w1'], p['b1'].reshape(1, H), p['w2'], p['b2'].reshape(1, H),
      p['g'].reshape(1, H), p['beta'].reshape(1, H),
      w1_0[:H], w1_0[H:2 * H])


def _edge_enc(pc, pr, p):
    return pl.pallas_call(
        _edge_enc_body,
        grid=(E // BE,),
        in_specs=[pl.BlockSpec((BE, 16), lambda i: (i, 0)),
                  pl.BlockSpec((BE, 16), lambda i: (i, 0)),
                  _wblk((3, H)), _wblk((1, H)), _wblk((H, H)),
                  _wblk((1, H)), _wblk((1, H)), _wblk((1, H))],
        out_specs=_eblk(),
        out_shape=jax.ShapeDtypeStruct((E, H), jnp.float32),
        interpret=_INTERPRET,
    )(pc, pr, p['w1'], p['b1'].reshape(1, H), p['w2'],
      p['b2'].reshape(1, H), p['g'].reshape(1, H), p['beta'].reshape(1, H))


def kernel(x, pos, params, edge_index):
    row = edge_index[0]
    col = edge_index[1]
    xP = jnp.pad(x, ((0, NP - N), (0, 0)))
    posP = jnp.pad(pos, ((0, NP - N), (0, 14)))

    pc = posP[col]
    pr = posP[row]
    e = _edge_enc(pc, pr, params['edge_enc'])

    h, hA, hB = _enc_nodes(xP, params['node_enc'],
                           params['layers'][0]['edge']['w1'])

    n_layers = len(params['layers'])
    out = None
    for l in range(n_layers):
        lp = params['layers'][l]
        gA = hA[col]
        gB = hB[row]
        ue = _edge_mlp(gA, gB, e, lp['edge'])
        agg = jnp.zeros((NP, H), jnp.float32).at[row].add(ue)
        if l + 1 < n_layers:
            w1n = params['layers'][l + 1]['edge']['w1']
            h, hA, hB = _node_update(h, agg, lp['node'], w1n)
        else:
            out = _node_last(h, agg, lp['node'], params['decoder'])
        e = ue
    return out[:N, :2]


# speedup vs baseline: 2.4280x; 2.0136x over previous
import functools

import jax
from jax import lax
import jax.numpy as jnp
from jax.experimental import pallas as pl
from jax.experimental.pallas import tpu as pltpu
from jax.experimental.pallas import tpu_sc as plsc

N = 10000
E = 320000
H = 128
NP = 10240
BE = 2000
BN = 1024

SC_NC = 2
SC_NS = 16
NW = SC_NC * SC_NS
EPW = E // NW
CH = 80
NCH = EPW // CH

_INTERPRET = False


def _sc_mesh():
    return plsc.VectorSubcoreMesh(core_axis_name="c", subcore_axis_name="s",
                                  num_cores=SC_NC, num_subcores=SC_NS)


def _gather_pair(tblA, tblB, col3, row3):
    D = tblA.shape[1]

    @functools.partial(
        pl.kernel,
        out_type=[jax.ShapeDtypeStruct((E, D), jnp.float32)] * 2,
        mesh=_sc_mesh(),
        scratch_types=[
            pltpu.VMEM((NCH, CH), jnp.int32),
            pltpu.VMEM((NCH, CH), jnp.int32),
            pltpu.VMEM((CH, D), jnp.float32),
            pltpu.VMEM((CH, D), jnp.float32),
            pltpu.SemaphoreType.DMA,
            pltpu.SemaphoreType.DMA,
        ],
    )
    def k(tA, tB, c3, r3, gA, gB, cv, rv, bA, bB, s1, s2):
        wid = lax.axis_index("s") * SC_NC + lax.axis_index("c")
        pltpu.sync_copy(c3.at[wid], cv)
        pltpu.sync_copy(r3.at[wid], rv)

        def body(c, carry):
            base = wid * EPW + c * CH
            cpA = pltpu.async_copy(tA.at[cv.at[c]], bA, s1)
            cpB = pltpu.async_copy(tB.at[rv.at[c]], bB, s2)
            cpA.wait()
            cpB.wait()
            pltpu.sync_copy(bA, gA.at[pl.ds(base, CH)])
            pltpu.sync_copy(bB, gB.at[pl.ds(base, CH)])
            return carry

        lax.fori_loop(0, NCH, body, 0)

    return k(tblA, tblB, col3, row3)


def _scatter_add(ue, row3):
    @functools.partial(
        pl.kernel,
        out_type=jax.ShapeDtypeStruct((SC_NC, NP, H), jnp.float32),
        mesh=_sc_mesh(),
        scratch_types=[
            pltpu.VMEM((NCH, CH), jnp.int32),
            pltpu.VMEM((CH, H), jnp.float32),
            pltpu.VMEM_SHARED((NP, H), jnp.float32),
            pltpu.SemaphoreType.DMA,
        ],
    )
    def k(ue_h, r3, agg_h, rv, buf, acc, sem):
        cid = lax.axis_index("c")
        sid = lax.axis_index("s")
        wid = sid * SC_NC + cid
        pltpu.sync_copy(r3.at[wid], rv)

        def zrow(r, carry):
            for j in range(H // 16):
                buf[r, pl.ds(j * 16, 16)] = jnp.zeros((16,), jnp.float32)
            return carry

        lax.fori_loop(0, CH, zrow, 0)
        rows_per_sub = NP // SC_NS
        for kchunk in range(rows_per_sub // CH):
            off = sid * rows_per_sub + kchunk * CH
            pltpu.sync_copy(buf, acc.at[pl.ds(off, CH)])
        plsc.subcore_barrier()

        def body(c, carry):
            base = wid * EPW + c * CH
            pltpu.sync_copy(ue_h.at[pl.ds(base, CH)], buf)
            pltpu.sync_copy(buf, acc.at[rv.at[c]], add=True)
            return carry

        lax.fori_loop(0, NCH, body, 0)
        plsc.subcore_barrier()

        for kchunk in range(rows_per_sub // CH):
            off = sid * rows_per_sub + kchunk * CH
            pltpu.sync_copy(acc.at[pl.ds(off, CH)], buf)
            pltpu.sync_copy(buf, agg_h.at[cid, pl.ds(off, CH)])

    return k(ue, row3)


def _ln(u, g, beta):
    mu = jnp.mean(u, axis=-1, keepdims=True)
    var = jnp.mean((u - mu) * (u - mu), axis=-1, keepdims=True)
    return (u - mu) * jax.lax.rsqrt(var + 1e-5) * g + beta


def _dot(a, b):
    return jnp.dot(a, b, preferred_element_type=jnp.float32,
                   precision=lax.Precision.HIGHEST)


def _edge_mlp_body(gA, gB, e, w1c, b1, w2, b2, g, beta, out):
    pre = (gA[...][:, :H] + gB[...][:, :H] + _dot(e[...], w1c[...])
           + b1[...])
    t = jnp.maximum(pre, 0.0)
    u = _dot(t, w2[...]) + b2[...]
    out[...] = _ln(u, g[...], beta[...]) + e[...]


def _node_body(h, agg2, wa, wb, b1, w2, b2, g, beta, wna, wnb,
               hout, haout, hbout):
    agg = agg2[0] + agg2[1]
    pre = _dot(h[...], wa[...]) + _dot(agg, wb[...]) + b1[...]
    t = jnp.maximum(pre, 0.0)
    u = _dot(t, w2[...]) + b2[...]
    hn = h[...] + _ln(u, g[...], beta[...])
    hout[...] = hn
    haout[...] = _dot(hn, wna[...])
    hbout[...] = _dot(hn, wnb[...])


def _node_last_body(h, agg2, wa, wb, b1, w2, b2, g, beta,
                    wd1, bd1, wd2, bd2, out):
    agg = agg2[0] + agg2[1]
    pre = _dot(h[...], wa[...]) + _dot(agg, wb[...]) + b1[...]
    t = jnp.maximum(pre, 0.0)
    u = _dot(t, w2[...]) + b2[...]
    hn = h[...] + _ln(u, g[...], beta[...])
    d = jnp.maximum(_dot(hn, wd1[...]) + bd1[...], 0.0)
    out[...] = _dot(d, wd2[...]) + bd2[...]


def _enc_nodes_body(x, w1, b1, w2, b2, g, beta, wna, wnb,
                    hout, haout, hbout):
    pre = _dot(x[...], w1[...]) + b1[...]
    t = jnp.maximum(pre, 0.0)
    u = _dot(t, w2[...]) + b2[...]
    hn = _ln(u, g[...], beta[...])
    hout[...] = hn
    haout[...] = _dot(hn, wna[...])
    hbout[...] = _dot(hn, wnb[...])


def _edge_enc_body(pc, pr, w1, b1, w2, b2, g, beta, out):
    d = pc[...][:, 0:16] - pr[...][:, 0:16]
    r0 = d[:, 0:1]
    r1 = d[:, 1:2]
    dist = jnp.sqrt(r0 * r0 + r1 * r1)
    w = w1[...]
    pre = r0 * w[0:1, :] + r1 * w[1:2, :] + dist * w[2:3, :] + b1[...]
    t = jnp.maximum(pre, 0.0)
    u = _dot(t, w2[...]) + b2[...]
    out[...] = _ln(u, g[...], beta[...])


def _eblk():
    return pl.BlockSpec((BE, H), lambda i: (i, 0))


def _nblk():
    return pl.BlockSpec((BN, H), lambda i: (i, 0))


def _wblk(shape):
    return pl.BlockSpec(shape, lambda i: tuple(0 for _ in shape))


def _edge_mlp(gA, gB, e, p):
    DG = gA.shape[1]
    gblk = pl.BlockSpec((BE, DG), lambda i: (i, 0))
    return pl.pallas_call(
        _edge_mlp_body,
        grid=(E // BE,),
        in_specs=[gblk, gblk, _eblk(),
                  _wblk((H, H)), _wblk((1, H)), _wblk((H, H)),
                  _wblk((1, H)), _wblk((1, H)), _wblk((1, H))],
        out_specs=_eblk(),
        out_shape=jax.ShapeDtypeStruct((E, H), jnp.float32),
        interpret=_INTERPRET,
    )(gA, gB, e, p['w1'][2 * H:], p['b1'].reshape(1, H), p['w2'],
      p['b2'].reshape(1, H), p['g'].reshape(1, H), p['beta'].reshape(1, H))


def _aggblk():
    return pl.BlockSpec((SC_NC, BN, H), lambda i: (0, i, 0))


def _node_update(h, agg2, p, w1_next):
    outs = [jax.ShapeDtypeStruct((NP, H), jnp.float32)] * 3
    return pl.pallas_call(
        _node_body,
        grid=(NP // BN,),
        in_specs=[_nblk(), _aggblk(),
                  _wblk((H, H)), _wblk((H, H)), _wblk((1, H)),
                  _wblk((H, H)), _wblk((1, H)), _wblk((1, H)), _wblk((1, H)),
                  _wblk((H, H)), _wblk((H, H))],
        out_specs=[_nblk()] * 3,
        out_shape=outs,
        interpret=_INTERPRET,
    )(h, agg2, p['w1'][:H], p['w1'][H:], p['b1'].reshape(1, H), p['w2'],
      p['b2'].reshape(1, H), p['g'].reshape(1, H), p['beta'].reshape(1, H),
      w1_next[:H], w1_next[H:2 * H])


def _node_last(h, agg2, p, dec):
    wd2 = jnp.pad(dec['w2'], ((0, 0), (0, H - dec['w2'].shape[1])))
    bd2 = jnp.pad(dec['b2'], (0, H - dec['b2'].shape[0])).reshape(1, H)
    return pl.pallas_call(
        _node_last_body,
        grid=(NP // BN,),
        in_specs=[_nblk(), _aggblk(),
                  _wblk((H, H)), _wblk((H, H)), _wblk((1, H)),
                  _wblk((H, H)), _wblk((1, H)), _wblk((1, H)), _wblk((1, H)),
                  _wblk((H, H)), _wblk((1, H)), _wblk((H, H)), _wblk((1, H))],
        out_specs=_nblk(),
        out_shape=jax.ShapeDtypeStruct((NP, H), jnp.float32),
        interpret=_INTERPRET,
    )(h, agg2, p['w1'][:H], p['w1'][H:], p['b1'].reshape(1, H), p['w2'],
      p['b2'].reshape(1, H), p['g'].reshape(1, H), p['beta'].reshape(1, H),
      dec['w1'], dec['b1'].reshape(1, H), wd2, bd2)


def _enc_nodes(xP, p, w1_0):
    outs = [jax.ShapeDtypeStruct((NP, H), jnp.float32)] * 3
    return pl.pallas_call(
        _enc_nodes_body,
        grid=(NP // BN,),
        in_specs=[pl.BlockSpec((BN, 4), lambda i: (i, 0)),
                  _wblk((4, H)), _wblk((1, H)), _wblk((H, H)),
                  _wblk((1, H)), _wblk((1, H)), _wblk((1, H)),
                  _wblk((H, H)), _wblk((H, H))],
        out_specs=[_nblk()] * 3,
        out_shape=outs,
        interpret=_INTERPRET,
    )(xP, p['w1'], p['b1'].reshape(1, H), p['w2'], p['b2'].reshape(1, H),
      p['g'].reshape(1, H), p['beta'].reshape(1, H),
      w1_0[:H], w1_0[H:2 * H])


def _edge_enc(g144A, g144B, p):
    gblk = pl.BlockSpec((BE, H), lambda i: (i, 0))
    return pl.pallas_call(
        _edge_enc_body,
        grid=(E // BE,),
        in_specs=[gblk, gblk,
                  _wblk((3, H)), _wblk((1, H)), _wblk((H, H)),
                  _wblk((1, H)), _wblk((1, H)), _wblk((1, H))],
        out_specs=_eblk(),
        out_shape=jax.ShapeDtypeStruct((E, H), jnp.float32),
        interpret=_INTERPRET,
    )(g144A, g144B, p['w1'], p['b1'].reshape(1, H), p['w2'],
      p['b2'].reshape(1, H), p['g'].reshape(1, H), p['beta'].reshape(1, H))


def kernel(x, pos, params, edge_index):
    row = edge_index[0]
    col = edge_index[1]
    col3 = col.reshape(NW, NCH, CH)
    row3 = row.reshape(NW, NCH, CH)
    xP = jnp.pad(x, ((0, NP - N), (0, 0)))
    posP = jnp.pad(pos, ((0, NP - N), (0, H - pos.shape[1])))

    h, hA, hB = _enc_nodes(xP, params['node_enc'],
                           params['layers'][0]['edge']['w1'])
    pc, pr = _gather_pair(posP, posP, col3, row3)
    e = _edge_enc(pc, pr, params['edge_enc'])

    n_layers = len(params['layers'])
    out = None
    for l in range(n_layers):
        lp = params['layers'][l]
        gA, gB = _gather_pair(hA, hB, col3, row3)
        ue = _edge_mlp(gA, gB, e, lp['edge'])
        agg2 = _scatter_add(ue, row3)
        if l + 1 < n_layers:
            w1n = params['layers'][l + 1]['edge']['w1']
            h, hA, hB = _node_update(h, agg2, lp['node'], w1n)
        else:
            out = _node_last(h, agg2, lp['node'], params['decoder'])
        e = ue
    return out[:N, :2]


# speedup vs baseline: 3.9009x; 1.6066x over previous
import functools

import jax
from jax import lax
import jax.numpy as jnp
from jax.experimental import pallas as pl
from jax.experimental.pallas import tpu as pltpu
from jax.experimental.pallas import tpu_sc as plsc

N = 10000
E = 320000
H = 128
NP = 10240
BE = 2000
BN = 1024

SC_NC = 2
SC_NS = 16
NW = SC_NC * SC_NS
EPW = E // NW
CH = 80
NCH = EPW // CH

_INTERPRET = False


def _sc_mesh():
    return plsc.VectorSubcoreMesh(core_axis_name="c", subcore_axis_name="s",
                                  num_cores=SC_NC, num_subcores=SC_NS)


def _gather_pair(tblA, tblB, col3, row3):
    D = tblA.shape[1]

    @functools.partial(
        pl.kernel,
        out_type=[jax.ShapeDtypeStruct((E, D), jnp.float32)] * 2,
        mesh=_sc_mesh(),
        scratch_types=[
            pltpu.VMEM((NCH, CH), jnp.int32),
            pltpu.VMEM((NCH, CH), jnp.int32),
            pltpu.VMEM((CH, D), jnp.float32),
            pltpu.VMEM((CH, D), jnp.float32),
            pltpu.VMEM((CH, D), jnp.float32),
            pltpu.VMEM((CH, D), jnp.float32),
        ] + [pltpu.SemaphoreType.DMA] * 8,
    )
    def k(tA, tB, c3, r3, gA, gB, cv, rv,
          bA0, bA1, bB0, bB1, gsA0, gsA1, gsB0, gsB1,
          ssA0, ssA1, ssB0, ssB1):
        wid = lax.axis_index("s") * SC_NC + lax.axis_index("c")
        bufA = (bA0, bA1)
        bufB = (bB0, bB1)
        gsA = (gsA0, gsA1)
        gsB = (gsB0, gsB1)
        ssA = (ssA0, ssA1)
        ssB = (ssB0, ssB1)
        pltpu.sync_copy(c3.at[wid], cv)
        pltpu.sync_copy(r3.at[wid], rv)

        def gstart(c, b):
            pltpu.async_copy(tA.at[cv.at[c]], bufA[b], gsA[b])
            pltpu.async_copy(tB.at[rv.at[c]], bufB[b], gsB[b])

        def gwait(c, b):
            pltpu.make_async_copy(tA.at[cv.at[c]], bufA[b], gsA[b]).wait()
            pltpu.make_async_copy(tB.at[rv.at[c]], bufB[b], gsB[b]).wait()

        def sstart(c, b):
            base = wid * EPW + c * CH
            pltpu.async_copy(bufA[b], gA.at[pl.ds(base, CH)], ssA[b])
            pltpu.async_copy(bufB[b], gB.at[pl.ds(base, CH)], ssB[b])

        def swait(c, b):
            base = wid * EPW + c * CH
            pltpu.make_async_copy(bufA[b], gA.at[pl.ds(base, CH)],
                                  ssA[b]).wait()
            pltpu.make_async_copy(bufB[b], gB.at[pl.ds(base, CH)],
                                  ssB[b]).wait()

        gstart(0, 0)

        def body(gidx, carry):
            for b in range(2):
                c = 2 * gidx + b
                gwait(c, b)

                @pl.when(c > 0)
                def _():
                    swait(c - 1, 1 - b)

                @pl.when(c + 1 < NCH)
                def _():
                    gstart(c + 1, 1 - b)

                sstart(c, b)
            return carry

        lax.fori_loop(0, NCH // 2, body, 0)
        if NCH % 2 == 1:
            c = NCH - 1
            gwait(c, 0)
            swait(c - 1, 1)
            sstart(c, 0)
            swait(c, 0)
        else:
            swait(NCH - 2, 0)
            swait(NCH - 1, 1)

    return k(tblA, tblB, col3, row3)


def _scatter_add(ue, row3):
    ZCH = 80

    @functools.partial(
        pl.kernel,
        out_type=jax.ShapeDtypeStruct((SC_NC, NP, H), jnp.float32),
        mesh=_sc_mesh(),
        scratch_types=[
            pltpu.VMEM((NCH, CH), jnp.int32),
            pltpu.VMEM((ZCH, H), jnp.float32),
            pltpu.VMEM((CH, H), jnp.float32),
            pltpu.VMEM((CH, H), jnp.float32),
            pltpu.VMEM_SHARED((NP, H), jnp.float32),
        ] + [pltpu.SemaphoreType.DMA] * 4,
    )
    def k(ue_h, r3, agg_h, rv, zbuf, b0, b1, acc, ls0, ls1, as0, as1):
        cid = lax.axis_index("c")
        sid = lax.axis_index("s")
        wid = sid * SC_NC + cid
        bufs = (b0, b1)
        lsem = (ls0, ls1)
        asem = (as0, as1)
        pltpu.sync_copy(r3.at[wid], rv)

        def zrow(r, carry):
            for j in range(H // 16):
                zbuf[r, pl.ds(j * 16, 16)] = jnp.zeros((16,), jnp.float32)
            return carry

        lax.fori_loop(0, ZCH, zrow, 0)
        rows_per_sub = NP // SC_NS
        for kchunk in range(rows_per_sub // ZCH):
            off = sid * rows_per_sub + kchunk * ZCH
            pltpu.sync_copy(zbuf, acc.at[pl.ds(off, ZCH)])
        plsc.subcore_barrier()

        def lstart(c, b):
            base = wid * EPW + c * CH
            pltpu.async_copy(ue_h.at[pl.ds(base, CH)], bufs[b], lsem[b])

        def lwait(c, b):
            base = wid * EPW + c * CH
            pltpu.make_async_copy(ue_h.at[pl.ds(base, CH)], bufs[b],
                                  lsem[b]).wait()

        def astart(c, b):
            pltpu.async_copy(bufs[b], acc.at[rv.at[c]], asem[b], add=True)

        def await_(c, b):
            pltpu.make_async_copy(bufs[b], acc.at[rv.at[c]],
                                  asem[b]).wait()

        lstart(0, 0)

        def body(gidx, carry):
            for b in range(2):
                c = 2 * gidx + b
                lwait(c, b)

                @pl.when(c > 0)
                def _():
                    await_(c - 1, 1 - b)

                @pl.when(c + 1 < NCH)
                def _():
                    lstart(c + 1, 1 - b)

                astart(c, b)
            return carry

        lax.fori_loop(0, NCH // 2, body, 0)
        if NCH % 2 == 1:
            c = NCH - 1
            lwait(c, 0)
            await_(c - 1, 1)
            astart(c, 0)
            await_(c, 0)
        else:
            await_(NCH - 2, 0)
            await_(NCH - 1, 1)
        plsc.subcore_barrier()

        for kchunk in range(rows_per_sub // ZCH):
            off = sid * rows_per_sub + kchunk * ZCH
            pltpu.sync_copy(acc.at[pl.ds(off, ZCH)], zbuf)
            pltpu.sync_copy(zbuf, agg_h.at[cid, pl.ds(off, ZCH)])

    return k(ue, row3)


def _ln(u, g, beta):
    mu = jnp.mean(u, axis=-1, keepdims=True)
    var = jnp.mean((u - mu) * (u - mu), axis=-1, keepdims=True)
    return (u - mu) * jax.lax.rsqrt(var + 1e-5) * g + beta


def _dot(a, b):
    return jnp.dot(a.astype(jnp.bfloat16), b.astype(jnp.bfloat16),
                   preferred_element_type=jnp.float32)


def _edge_mlp_body(gA, gB, e, w1c, b1, w2, b2, g, beta, out):
    pre = (gA[...][:, :H] + gB[...][:, :H] + _dot(e[...], w1c[...])
           + b1[...])
    t = jnp.maximum(pre, 0.0)
    u = _dot(t, w2[...]) + b2[...]
    out[...] = _ln(u, g[...], beta[...]) + e[...]


def _node_body(h, agg2, wa, wb, b1, w2, b2, g, beta, wna, wnb,
               hout, haout, hbout):
    agg = agg2[0] + agg2[1]
    pre = _dot(h[...], wa[...]) + _dot(agg, wb[...]) + b1[...]
    t = jnp.maximum(pre, 0.0)
    u = _dot(t, w2[...]) + b2[...]
    hn = h[...] + _ln(u, g[...], beta[...])
    hout[...] = hn
    haout[...] = _dot(hn, wna[...])
    hbout[...] = _dot(hn, wnb[...])


def _node_last_body(h, agg2, wa, wb, b1, w2, b2, g, beta,
                    wd1, bd1, wd2, bd2, out):
    agg = agg2[0] + agg2[1]
    pre = _dot(h[...], wa[...]) + _dot(agg, wb[...]) + b1[...]
    t = jnp.maximum(pre, 0.0)
    u = _dot(t, w2[...]) + b2[...]
    hn = h[...] + _ln(u, g[...], beta[...])
    d = jnp.maximum(_dot(hn, wd1[...]) + bd1[...], 0.0)
    out[...] = _dot(d, wd2[...]) + bd2[...]


def _enc_nodes_body(x, w1, b1, w2, b2, g, beta, wna, wnb,
                    hout, haout, hbout):
    pre = _dot(x[...], w1[...]) + b1[...]
    t = jnp.maximum(pre, 0.0)
    u = _dot(t, w2[...]) + b2[...]
    hn = _ln(u, g[...], beta[...])
    hout[...] = hn
    haout[...] = _dot(hn, wna[...])
    hbout[...] = _dot(hn, wnb[...])


def _edge_enc_body(pc, pr, w1, b1, w2, b2, g, beta, out):
    d = pc[...][:, 0:16] - pr[...][:, 0:16]
    r0 = d[:, 0:1]
    r1 = d[:, 1:2]
    dist = jnp.sqrt(r0 * r0 + r1 * r1)
    ea = jnp.concatenate([r0, r1, dist], axis=1)
    pre = _dot(ea, w1[...]) + b1[...]
    t = jnp.maximum(pre, 0.0)
    u = _dot(t, w2[...]) + b2[...]
    out[...] = _ln(u, g[...], beta[...])


def _eblk():
    return pl.BlockSpec((BE, H), lambda i: (i, 0))


def _nblk():
    return pl.BlockSpec((BN, H), lambda i: (i, 0))


def _wblk(shape):
    return pl.BlockSpec(shape, lambda i: tuple(0 for _ in shape))


def _edge_mlp(gA, gB, e, p):
    DG = gA.shape[1]
    gblk = pl.BlockSpec((BE, DG), lambda i: (i, 0))
    return pl.pallas_call(
        _edge_mlp_body,
        grid=(E // BE,),
        in_specs=[gblk, gblk, _eblk(),
                  _wblk((H, H)), _wblk((1, H)), _wblk((H, H)),
                  _wblk((1, H)), _wblk((1, H)), _wblk((1, H))],
        out_specs=_eblk(),
        out_shape=jax.ShapeDtypeStruct((E, H), jnp.float32),
        interpret=_INTERPRET,
    )(gA, gB, e, p['w1'][2 * H:], p['b1'].reshape(1, H), p['w2'],
      p['b2'].reshape(1, H), p['g'].reshape(1, H), p['beta'].reshape(1, H))


def _aggblk():
    return pl.BlockSpec((SC_NC, BN, H), lambda i: (0, i, 0))


def _node_update(h, agg2, p, w1_next):
    outs = [jax.ShapeDtypeStruct((NP, H), jnp.float32)] * 3
    return pl.pallas_call(
        _node_body,
        grid=(NP // BN,),
        in_specs=[_nblk(), _aggblk(),
                  _wblk((H, H)), _wblk((H, H)), _wblk((1, H)),
                  _wblk((H, H)), _wblk((1, H)), _wblk((1, H)), _wblk((1, H)),
                  _wblk((H, H)), _wblk((H, H))],
        out_specs=[_nblk()] * 3,
        out_shape=outs,
        interpret=_INTERPRET,
    )(h, agg2, p['w1'][:H], p['w1'][H:], p['b1'].reshape(1, H), p['w2'],
      p['b2'].reshape(1, H), p['g'].reshape(1, H), p['beta'].reshape(1, H),
      w1_next[:H], w1_next[H:2 * H])


def _node_last(h, agg2, p, dec):
    wd2 = jnp.pad(dec['w2'], ((0, 0), (0, H - dec['w2'].shape[1])))
    bd2 = jnp.pad(dec['b2'], (0, H - dec['b2'].shape[0])).reshape(1, H)
    return pl.pallas_call(
        _node_last_body,
        grid=(NP // BN,),
        in_specs=[_nblk(), _aggblk(),
                  _wblk((H, H)), _wblk((H, H)), _wblk((1, H)),
                  _wblk((H, H)), _wblk((1, H)), _wblk((1, H)), _wblk((1, H)),
                  _wblk((H, H)), _wblk((1, H)), _wblk((H, H)), _wblk((1, H))],
        out_specs=_nblk(),
        out_shape=jax.ShapeDtypeStruct((NP, H), jnp.float32),
        interpret=_INTERPRET,
    )(h, agg2, p['w1'][:H], p['w1'][H:], p['b1'].reshape(1, H), p['w2'],
      p['b2'].reshape(1, H), p['g'].reshape(1, H), p['beta'].reshape(1, H),
      dec['w1'], dec['b1'].reshape(1, H), wd2, bd2)


def _enc_nodes(xP, p, w1_0):
    outs = [jax.ShapeDtypeStruct((NP, H), jnp.float32)] * 3
    return pl.pallas_call(
        _enc_nodes_body,
        grid=(NP // BN,),
        in_specs=[pl.BlockSpec((BN, 4), lambda i: (i, 0)),
                  _wblk((4, H)), _wblk((1, H)), _wblk((H, H)),
                  _wblk((1, H)), _wblk((1, H)), _wblk((1, H)),
                  _wblk((H, H)), _wblk((H, H))],
        out_specs=[_nblk()] * 3,
        out_shape=outs,
        interpret=_INTERPRET,
    )(xP, p['w1'], p['b1'].reshape(1, H), p['w2'], p['b2'].reshape(1, H),
      p['g'].reshape(1, H), p['beta'].reshape(1, H),
      w1_0[:H], w1_0[H:2 * H])


def _edge_enc(g144A, g144B, p):
    gblk = pl.BlockSpec((BE, H), lambda i: (i, 0))
    return pl.pallas_call(
        _edge_enc_body,
        grid=(E // BE,),
        in_specs=[gblk, gblk,
                  _wblk((3, H)), _wblk((1, H)), _wblk((H, H)),
                  _wblk((1, H)), _wblk((1, H)), _wblk((1, H))],
        out_specs=_eblk(),
        out_shape=jax.ShapeDtypeStruct((E, H), jnp.float32),
        interpret=_INTERPRET,
    )(g144A, g144B, p['w1'], p['b1'].reshape(1, H), p['w2'],
      p['b2'].reshape(1, H), p['g'].reshape(1, H), p['beta'].reshape(1, H))


def kernel(x, pos, params, edge_index):
    row = edge_index[0]
    col = edge_index[1]
    col3 = col.reshape(NW, NCH, CH)
    row3 = row.reshape(NW, NCH, CH)
    xP = jnp.pad(x, ((0, NP - N), (0, 0)))
    posP = jnp.pad(pos, ((0, NP - N), (0, H - pos.shape[1])))

    h, hA, hB = _enc_nodes(xP, params['node_enc'],
                           params['layers'][0]['edge']['w1'])
    pc, pr = _gather_pair(posP, posP, col3, row3)
    e = _edge_enc(pc, pr, params['edge_enc'])

    n_layers = len(params['layers'])
    out = None
    for l in range(n_layers):
        lp = params['layers'][l]
        gA, gB = _gather_pair(hA, hB, col3, row3)
        ue = _edge_mlp(gA, gB, e, lp['edge'])
        agg2 = _scatter_add(ue, row3)
        if l + 1 < n_layers:
            w1n = params['layers'][l + 1]['edge']['w1']
            h, hA, hB = _node_update(h, agg2, lp['node'], w1n)
        else:
            out = _node_last(h, agg2, lp['node'], params['decoder'])
        e = ue
    return out[:N, :2]


# speedup vs baseline: 4.3406x; 1.1127x over previous
import functools

import jax
from jax import lax
import jax.numpy as jnp
from jax.experimental import pallas as pl
from jax.experimental.pallas import tpu as pltpu
from jax.experimental.pallas import tpu_sc as plsc

N = 10000
E = 320000
H = 128
NP = 10240
BE = 2000
BN = 1024

SC_NC = 2
SC_NS = 16
NW = SC_NC * SC_NS
CH = 80
SB = (192000, 128000)

_INTERPRET = False


def _sc_mesh():
    return plsc.VectorSubcoreMesh(core_axis_name="c", subcore_axis_name="s",
                                  num_cores=SC_NC, num_subcores=SC_NS)


def _gather_pair(tblA, tblB, col3, row3):
    D = tblA.shape[1]
    NCH = col3.shape[1]
    EPW = NCH * CH
    ESB = NW * EPW

    @functools.partial(
        pl.kernel,
        out_type=[jax.ShapeDtypeStruct((ESB, D), jnp.float32)] * 2,
        mesh=_sc_mesh(),
        scratch_types=[
            pltpu.VMEM((NCH, CH), jnp.int32),
            pltpu.VMEM((NCH, CH), jnp.int32),
            pltpu.VMEM((CH, D), jnp.float32),
            pltpu.VMEM((CH, D), jnp.float32),
            pltpu.VMEM((CH, D), jnp.float32),
            pltpu.VMEM((CH, D), jnp.float32),
        ] + [pltpu.SemaphoreType.DMA] * 8,
    )
    def k(tA, tB, c3, r3, gA, gB, cv, rv,
          bA0, bA1, bB0, bB1, gsA0, gsA1, gsB0, gsB1,
          ssA0, ssA1, ssB0, ssB1):
        wid = lax.axis_index("s") * SC_NC + lax.axis_index("c")
        bufA = (bA0, bA1)
        bufB = (bB0, bB1)
        gsA = (gsA0, gsA1)
        gsB = (gsB0, gsB1)
        ssA = (ssA0, ssA1)
        ssB = (ssB0, ssB1)
        pltpu.sync_copy(c3.at[wid], cv)
        pltpu.sync_copy(r3.at[wid], rv)

        def gstart(c, b):
            pltpu.async_copy(tA.at[cv.at[c]], bufA[b], gsA[b])
            pltpu.async_copy(tB.at[rv.at[c]], bufB[b], gsB[b])

        def gwait(c, b):
            pltpu.make_async_copy(tA.at[cv.at[c]], bufA[b], gsA[b]).wait()
            pltpu.make_async_copy(tB.at[rv.at[c]], bufB[b], gsB[b]).wait()

        def sstart(c, b):
            base = wid * EPW + c * CH
            pltpu.async_copy(bufA[b], gA.at[pl.ds(base, CH)], ssA[b])
            pltpu.async_copy(bufB[b], gB.at[pl.ds(base, CH)], ssB[b])

        def swait(c, b):
            base = wid * EPW + c * CH
            pltpu.make_async_copy(bufA[b], gA.at[pl.ds(base, CH)],
                                  ssA[b]).wait()
            pltpu.make_async_copy(bufB[b], gB.at[pl.ds(base, CH)],
                                  ssB[b]).wait()

        gstart(0, 0)

        def body(gidx, carry):
            for b in range(2):
                c = 2 * gidx + b
                gwait(c, b)

                @pl.when(c > 0)
                def _():
                    swait(c - 1, 1 - b)

                @pl.when(c + 1 < NCH)
                def _():
                    gstart(c + 1, 1 - b)

                sstart(c, b)
            return carry

        lax.fori_loop(0, NCH // 2, body, 0)
        if NCH % 2 == 1:
            c = NCH - 1
            gwait(c, 0)
            swait(c - 1, 1)
            sstart(c, 0)
            swait(c, 0)
        else:
            swait(NCH - 1, 1)

    return k(tblA, tblB, col3, row3)


def _scatter_add(ue, row3):
    ZCH = 80
    NCH = row3.shape[1]
    EPW = NCH * CH

    @functools.partial(
        pl.kernel,
        out_type=jax.ShapeDtypeStruct((SC_NC, NP, H), jnp.float32),
        mesh=_sc_mesh(),
        scratch_types=[
            pltpu.VMEM((NCH, CH), jnp.int32),
            pltpu.VMEM((ZCH, H), jnp.float32),
            pltpu.VMEM((CH, H), jnp.float32),
            pltpu.VMEM((CH, H), jnp.float32),
            pltpu.VMEM_SHARED((NP, H), jnp.float32),
        ] + [pltpu.SemaphoreType.DMA] * 4,
    )
    def k(ue_h, r3, agg_h, rv, zbuf, b0, b1, acc, ls0, ls1, as0, as1):
        cid = lax.axis_index("c")
        sid = lax.axis_index("s")
        wid = sid * SC_NC + cid
        bufs = (b0, b1)
        lsem = (ls0, ls1)
        asem = (as0, as1)
        pltpu.sync_copy(r3.at[wid], rv)

        def zrow(r, carry):
            for j in range(H // 16):
                zbuf[r, pl.ds(j * 16, 16)] = jnp.zeros((16,), jnp.float32)
            return carry

        lax.fori_loop(0, ZCH, zrow, 0)
        rows_per_sub = NP // SC_NS
        for kchunk in range(rows_per_sub // ZCH):
            off = sid * rows_per_sub + kchunk * ZCH
            pltpu.sync_copy(zbuf, acc.at[pl.ds(off, ZCH)])
        plsc.subcore_barrier()

        def lstart(c, b):
            base = wid * EPW + c * CH
            pltpu.async_copy(ue_h.at[pl.ds(base, CH)], bufs[b], lsem[b])

        def lwait(c, b):
            base = wid * EPW + c * CH
            pltpu.make_async_copy(ue_h.at[pl.ds(base, CH)], bufs[b],
                                  lsem[b]).wait()

        def astart(c, b):
            pltpu.async_copy(bufs[b], acc.at[rv.at[c]], asem[b], add=True)

        def await_(c, b):
            pltpu.make_async_copy(bufs[b], acc.at[rv.at[c]],
                                  asem[b]).wait()

        lstart(0, 0)

        def body(gidx, carry):
            for b in range(2):
                c = 2 * gidx + b
                lwait(c, b)

                @pl.when(c > 0)
                def _():
                    await_(c - 1, 1 - b)

                @pl.when(c + 1 < NCH)
                def _():
                    lstart(c + 1, 1 - b)

                astart(c, b)
            return carry

        lax.fori_loop(0, NCH // 2, body, 0)
        if NCH % 2 == 1:
            c = NCH - 1
            lwait(c, 0)
            await_(c - 1, 1)
            astart(c, 0)
            await_(c, 0)
        else:
            await_(NCH - 1, 1)
        plsc.subcore_barrier()

        for kchunk in range(rows_per_sub // ZCH):
            off = sid * rows_per_sub + kchunk * ZCH
            pltpu.sync_copy(acc.at[pl.ds(off, ZCH)], zbuf)
            pltpu.sync_copy(zbuf, agg_h.at[cid, pl.ds(off, ZCH)])

    return k(ue, row3)


def _ln(u, g, beta):
    mu = jnp.mean(u, axis=-1, keepdims=True)
    var = jnp.mean((u - mu) * (u - mu), axis=-1, keepdims=True)
    return (u - mu) * jax.lax.rsqrt(var + 1e-5) * g + beta


def _dot(a, b):
    return jnp.dot(a.astype(jnp.bfloat16), b.astype(jnp.bfloat16),
                   preferred_element_type=jnp.float32)


def _edge_mlp_body(gA, gB, e, w1c, b1, w2, b2, g, beta, out):
    pre = (gA[...][:, :H] + gB[...][:, :H] + _dot(e[...], w1c[...])
           + b1[...])
    t = jnp.maximum(pre, 0.0)
    u = _dot(t, w2[...]) + b2[...]
    out[...] = _ln(u, g[...], beta[...]) + e[...]


def _node_body(h, agg2, agg2b, wa, wb, b1, w2, b2, g, beta, wna, wnb,
               hout, haout, hbout):
    agg = (agg2[0] + agg2[1]) + (agg2b[0] + agg2b[1])
    pre = _dot(h[...], wa[...]) + _dot(agg, wb[...]) + b1[...]
    t = jnp.maximum(pre, 0.0)
    u = _dot(t, w2[...]) + b2[...]
    hn = h[...] + _ln(u, g[...], beta[...])
    hout[...] = hn
    haout[...] = _dot(hn, wna[...])
    hbout[...] = _dot(hn, wnb[...])


def _node_last_body(h, agg2, agg2b, wa, wb, b1, w2, b2, g, beta,
                    wd1, bd1, wd2, bd2, out):
    agg = (agg2[0] + agg2[1]) + (agg2b[0] + agg2b[1])
    pre = _dot(h[...], wa[...]) + _dot(agg, wb[...]) + b1[...]
    t = jnp.maximum(pre, 0.0)
    u = _dot(t, w2[...]) + b2[...]
    hn = h[...] + _ln(u, g[...], beta[...])
    d = jnp.maximum(_dot(hn, wd1[...]) + bd1[...], 0.0)
    out[...] = _dot(d, wd2[...]) + bd2[...]


def _enc_nodes_body(x, w1, b1, w2, b2, g, beta, wna, wnb,
                    hout, haout, hbout):
    pre = _dot(x[...], w1[...]) + b1[...]
    t = jnp.maximum(pre, 0.0)
    u = _dot(t, w2[...]) + b2[...]
    hn = _ln(u, g[...], beta[...])
    hout[...] = hn
    haout[...] = _dot(hn, wna[...])
    hbout[...] = _dot(hn, wnb[...])


def _edge_enc_body(pc, pr, w1, b1, w2, b2, g, beta, out):
    d = pc[...][:, 0:16] - pr[...][:, 0:16]
    r0 = d[:, 0:1]
    r1 = d[:, 1:2]
    dist = jnp.sqrt(r0 * r0 + r1 * r1)
    ea = jnp.concatenate([r0, r1, dist], axis=1)
    pre = _dot(ea, w1[...]) + b1[...]
    t = jnp.maximum(pre, 0.0)
    u = _dot(t, w2[...]) + b2[...]
    out[...] = _ln(u, g[...], beta[...])


def _eblk():
    return pl.BlockSpec((BE, H), lambda i: (i, 0))


def _nblk():
    return pl.BlockSpec((BN, H), lambda i: (i, 0))


def _wblk(shape):
    return pl.BlockSpec(shape, lambda i: tuple(0 for _ in shape))


def _edge_mlp(gA, gB, e, p):
    ESB = gA.shape[0]
    return pl.pallas_call(
        _edge_mlp_body,
        grid=(ESB // BE,),
        in_specs=[_eblk(), _eblk(), _eblk(),
                  _wblk((H, H)), _wblk((1, H)), _wblk((H, H)),
                  _wblk((1, H)), _wblk((1, H)), _wblk((1, H))],
        out_specs=_eblk(),
        out_shape=jax.ShapeDtypeStruct((ESB, H), jnp.float32),
        interpret=_INTERPRET,
    )(gA, gB, e, p['w1'][2 * H:], p['b1'].reshape(1, H), p['w2'],
      p['b2'].reshape(1, H), p['g'].reshape(1, H), p['beta'].reshape(1, H))


def _aggblk():
    return pl.BlockSpec((SC_NC, BN, H), lambda i: (0, i, 0))


def _node_update(h, agg2, agg2b, p, w1_next):
    outs = [jax.ShapeDtypeStruct((NP, H), jnp.float32)] * 3
    return pl.pallas_call(
        _node_body,
        grid=(NP // BN,),
        in_specs=[_nblk(), _aggblk(), _aggblk(),
                  _wblk((H, H)), _wblk((H, H)), _wblk((1, H)),
                  _wblk((H, H)), _wblk((1, H)), _wblk((1, H)), _wblk((1, H)),
                  _wblk((H, H)), _wblk((H, H))],
        out_specs=[_nblk()] * 3,
        out_shape=outs,
        interpret=_INTERPRET,
    )(h, agg2, agg2b, p['w1'][:H], p['w1'][H:], p['b1'].reshape(1, H), p['w2'],
      p['b2'].reshape(1, H), p['g'].reshape(1, H), p['beta'].reshape(1, H),
      w1_next[:H], w1_next[H:2 * H])


def _node_last(h, agg2, agg2b, p, dec):
    wd2 = jnp.pad(dec['w2'], ((0, 0), (0, H - dec['w2'].shape[1])))
    bd2 = jnp.pad(dec['b2'], (0, H - dec['b2'].shape[0])).reshape(1, H)
    return pl.pallas_call(
        _node_last_body,
        grid=(NP // BN,),
        in_specs=[_nblk(), _aggblk(), _aggblk(),
                  _wblk((H, H)), _wblk((H, H)), _wblk((1, H)),
                  _wblk((H, H)), _wblk((1, H)), _wblk((1, H)), _wblk((1, H)),
                  _wblk((H, H)), _wblk((1, H)), _wblk((H, H)), _wblk((1, H))],
        out_specs=_nblk(),
        out_shape=jax.ShapeDtypeStruct((NP, H), jnp.float32),
        interpret=_INTERPRET,
    )(h, agg2, agg2b, p['w1'][:H], p['w1'][H:], p['b1'].reshape(1, H), p['w2'],
      p['b2'].reshape(1, H), p['g'].reshape(1, H), p['beta'].reshape(1, H),
      dec['w1'], dec['b1'].reshape(1, H), wd2, bd2)


def _enc_nodes(xP, p, w1_0):
    outs = [jax.ShapeDtypeStruct((NP, H), jnp.float32)] * 3
    return pl.pallas_call(
        _enc_nodes_body,
        grid=(NP // BN,),
        in_specs=[pl.BlockSpec((BN, 4), lambda i: (i, 0)),
                  _wblk((4, H)), _wblk((1, H)), _wblk((H, H)),
                  _wblk((1, H)), _wblk((1, H)), _wblk((1, H)),
                  _wblk((H, H)), _wblk((H, H))],
        out_specs=[_nblk()] * 3,
        out_shape=outs,
        interpret=_INTERPRET,
    )(xP, p['w1'], p['b1'].reshape(1, H), p['w2'], p['b2'].reshape(1, H),
      p['g'].reshape(1, H), p['beta'].reshape(1, H),
      w1_0[:H], w1_0[H:2 * H])


def _edge_enc(pc, pr, p):
    ESB = pc.shape[0]
    return pl.pallas_call(
        _edge_enc_body,
        grid=(ESB // BE,),
        in_specs=[_eblk(), _eblk(),
                  _wblk((3, H)), _wblk((1, H)), _wblk((H, H)),
                  _wblk((1, H)), _wblk((1, H)), _wblk((1, H))],
        out_specs=_eblk(),
        out_shape=jax.ShapeDtypeStruct((ESB, H), jnp.float32),
        interpret=_INTERPRET,
    )(pc, pr, p['w1'], p['b1'].reshape(1, H), p['w2'],
      p['b2'].reshape(1, H), p['g'].reshape(1, H), p['beta'].reshape(1, H))


def kernel(x, pos, params, edge_index):
    row = edge_index[0]
    col = edge_index[1]
    col3s, row3s = [], []
    off = 0
    for esb in SB:
        col3s.append(lax.slice(col, (off,), (off + esb,))
                     .reshape(NW, esb // (NW * CH), CH))
        row3s.append(lax.slice(row, (off,), (off + esb,))
                     .reshape(NW, esb // (NW * CH), CH))
        off += esb
    xP = jnp.pad(x, ((0, NP - N), (0, 0)))
    posP = jnp.pad(pos, ((0, NP - N), (0, H - pos.shape[1])))

    h, hA, hB = _enc_nodes(xP, params['node_enc'],
                           params['layers'][0]['edge']['w1'])
    es = []
    for s in range(len(SB)):
        pc, pr = _gather_pair(posP, posP, col3s[s], row3s[s])
        es.append(_edge_enc(pc, pr, params['edge_enc']))

    n_layers = len(params['layers'])
    out = None
    for l in range(n_layers):
        lp = params['layers'][l]
        ues, aggs = [], []
        for s in range(len(SB)):
            gA, gB = _gather_pair(hA, hB, col3s[s], row3s[s])
            ue = _edge_mlp(gA, gB, es[s], lp['edge'])
            ues.append(ue)
            aggs.append(_scatter_add(ue, row3s[s]))
        if l + 1 < n_layers:
            w1n = params['layers'][l + 1]['edge']['w1']
            h, hA, hB = _node_update(h, aggs[0], aggs[1], lp['node'], w1n)
        else:
            out = _node_last(h, aggs[0], aggs[1], lp['node'],
                             params['decoder'])
        es = ues
    return out[:N, :2]
